# baseline (speedup 1.0000x reference)
import functools

import jax
import jax.numpy as jnp
import numpy as np
from jax import lax
from jax.experimental import pallas as pl
from jax.experimental.pallas import tpu as pltpu
from jax.experimental.pallas import tpu_sc as plsc

N = 10000
E = 320000
HEADS = 10
HID = 64
HP = 16
DE = 16
NG = 64
EPS = 1e-5
F32 = jnp.float32

NC = 2
NS = 16
NW = NC * NS
EPW = E // NW
CH = 80
NCHUNK = EPW // CH
NP = 10240
RPS = NP // NS

_mesh = plsc.VectorSubcoreMesh(
    core_axis_name="c", subcore_axis_name="s", num_cores=NC, num_subcores=NS)
_sc_params = pltpu.CompilerParams(use_tc_tiling_on_sc=False, needs_layout_passes=False)


def _worker():
    c = lax.axis_index("c")
    s = lax.axis_index("s")
    return c, s, s * NC + c


def _zero_shared(zb, accs, s):
    z16 = jnp.zeros((16,), F32)

    def zbody(i, _):
        w = zb.shape[1]
        for c0 in range(0, w, 16):
            zb[i, pl.ds(c0, 16)] = z16
        return 0

    lax.fori_loop(0, RPS, zbody, 0)
    for acc in accs:
        pltpu.sync_copy(zb, acc.at[pl.ds(pl.multiple_of(s * RPS, 8), RPS)])


CH0 = 40
NCH0 = EPW // CH0


@functools.partial(
    pl.kernel,
    out_type=(jax.ShapeDtypeStruct((NC, NP, DE), F32),
              jax.ShapeDtypeStruct((NC, NP, DE), F32)),
    mesh=_mesh,
    compiler_params=_sc_params,
    scratch_types=[
        pltpu.VMEM((2, CH0), jnp.int32),
        pltpu.VMEM((2, CH0, DE), F32),
        pltpu.VMEM((CH0, DE), F32),
        pltpu.VMEM((RPS, DE), F32),
        pltpu.VMEM_SHARED((NP, DE), F32),
        pltpu.VMEM_SHARED((NP, DE), F32),
        [pltpu.SemaphoreType.DMA] * 2,
    ],
)
def _sc_phase0(dst_hbm, ea_hbm, ls_out, cnt_out,
               dstv, eabuf, onesb, zb, acc_a, acc_b, semg):
    c, s, wid = _worker()
    one16 = jnp.ones((16,), F32)

    def obody(i, _):
        onesb[i] = one16
        return 0

    lax.fori_loop(0, CH0, obody, 0)
    _zero_shared(zb, (acc_a, acc_b), s)
    plsc.subcore_barrier()

    def base_of(ci):
        ci = lax.rem(ci, NCH0)
        return pl.multiple_of(wid * EPW + ci * CH0, CH0)

    def issue(b, ci):
        base = base_of(ci)
        pltpu.async_copy(dst_hbm.at[pl.ds(base, CH0)], dstv.at[b], semg[b])
        pltpu.async_copy(ea_hbm.at[pl.ds(base, CH0)], eabuf.at[b], semg[b])

    def wait(b):
        pltpu.make_async_copy(dst_hbm.at[pl.ds(0, CH0)], dstv.at[b], semg[b]).wait()
        pltpu.make_async_copy(ea_hbm.at[pl.ds(0, CH0)], eabuf.at[b], semg[b]).wait()

    def scat(b):
        pltpu.sync_copy(eabuf.at[b], acc_a.at[dstv.at[b]], add=True)
        pltpu.sync_copy(onesb, acc_b.at[dstv.at[b]], add=True)

    issue(0, 0)
    issue(1, 1)

    def pair(i, _):
        wait(0)
        scat(0)
        issue(0, 2 * i + 2)
        wait(1)
        scat(1)
        issue(1, 2 * i + 3)
        return 0

    lax.fori_loop(0, NCH0 // 2, pair, 0)
    wait(0)
    wait(1)
    plsc.subcore_barrier()
    off = pl.multiple_of(s * RPS, 8)
    pltpu.sync_copy(acc_a.at[pl.ds(off, RPS)], ls_out.at[c, pl.ds(off, RPS)])
    pltpu.sync_copy(acc_b.at[pl.ds(off, RPS)], cnt_out.at[c, pl.ds(off, RPS)])


CH2 = 40
NCH2 = EPW // CH2


@functools.partial(
    pl.kernel,
    out_type=(jax.ShapeDtypeStruct((E, HP), F32),
              jax.ShapeDtypeStruct((NC, NP, HP), F32)),
    mesh=_mesh,
    compiler_params=_sc_params,
    scratch_types=[
        pltpu.VMEM((2, CH2), jnp.int32),
        pltpu.VMEM((2, CH2), jnp.int32),
        pltpu.VMEM((2, CH2, HP), F32),
        pltpu.VMEM((2, CH2, HP), F32),
        pltpu.VMEM((2, CH2, HP), F32),
        pltpu.VMEM((2, CH2, HP), F32),
        pltpu.VMEM((RPS, HP), F32),
        pltpu.VMEM_SHARED((NP, HP), F32),
        [pltpu.SemaphoreType.DMA] * 2,
        [pltpu.SemaphoreType.DMA] * 2,
        [pltpu.SemaphoreType.DMA] * 2,
    ],
)
def _sc_pass2(src_hbm, dst_hbm, asn_hbm, adn_hbm, ae_hbm, ex_out, dpart_out,
              srcv, dstv, asb, adb, aeb, exb, zb, acc, semi, semg, semw):
    c, s, wid = _worker()
    _zero_shared(zb, (acc,), s)
    plsc.subcore_barrier()

    def base_of(ci):
        ci = lax.rem(ci, NCH2)
        return pl.multiple_of(wid * EPW + ci * CH2, CH2)

    def issue_idx(b, ci):
        base = base_of(ci)
        pltpu.async_copy(src_hbm.at[pl.ds(base, CH2)], srcv.at[b], semi[b])
        pltpu.async_copy(dst_hbm.at[pl.ds(base, CH2)], dstv.at[b], semi[b])

    def wait_idx(b):
        pltpu.make_async_copy(src_hbm.at[pl.ds(0, CH2)], srcv.at[b], semi[b]).wait()
        pltpu.make_async_copy(dst_hbm.at[pl.ds(0, CH2)], dstv.at[b], semi[b]).wait()

    def issue_g(b, ci):
        base = base_of(ci)
        pltpu.async_copy(asn_hbm.at[srcv.at[b]], asb.at[b], semg[b])
        pltpu.async_copy(adn_hbm.at[dstv.at[b]], adb.at[b], semg[b])
        pltpu.async_copy(ae_hbm.at[pl.ds(base, CH2)], aeb.at[b], semg[b])

    def wait_g(b):
        pltpu.make_async_copy(asn_hbm.at[srcv.at[b]], asb.at[b], semg[b]).wait()
        pltpu.make_async_copy(adn_hbm.at[dstv.at[b]], adb.at[b], semg[b]).wait()
        pltpu.make_async_copy(ae_hbm.at[pl.ds(0, CH2)], aeb.at[b], semg[b]).wait()

    def wait_w(b):
        pltpu.make_async_copy(exb.at[b], ex_out.at[pl.ds(0, CH2)], semw[b]).wait()

    def compute(b, ci):
        wait_w(b)

        @plsc.parallel_loop(0, CH2, 1, unroll=4)
        def ebody(e):
            a = asb[b, e] + adb[b, e] + aeb[b, e]
            a = jnp.where(a > 0.0, a, 0.2 * a)
            exb[b, e] = jnp.exp(a)
        base = base_of(ci)
        pltpu.async_copy(exb.at[b], ex_out.at[pl.ds(base, CH2)], semw[b])
        pltpu.sync_copy(exb.at[b], acc.at[dstv.at[b]], add=True)

    for b in range(2):
        pltpu.async_copy(ae_hbm.at[pl.ds(0, CH2)], exb.at[b], semw[b])
    issue_idx(0, 0)
    issue_idx(1, 1)
    wait_idx(0)
    issue_g(0, 0)

    def pair(i, _):
        a_ci = 2 * i
        wait_idx(1)
        issue_g(1, a_ci + 1)
        wait_g(0)
        compute(0, a_ci)
        issue_idx(0, a_ci + 2)
        wait_idx(0)
        issue_g(0, a_ci + 2)
        wait_g(1)
        compute(1, a_ci + 1)
        issue_idx(1, a_ci + 3)
        return 0

    lax.fori_loop(0, NCH2 // 2, pair, 0)
    wait_idx(1)
    wait_g(0)
    wait_w(0)
    wait_w(1)
    plsc.subcore_barrier()
    off = pl.multiple_of(s * RPS, 8)
    pltpu.sync_copy(acc.at[pl.ds(off, RPS)], dpart_out.at[c, pl.ds(off, RPS)])


HH = HID // 2
EPS3 = E // NS
NCH3 = EPS3 // CH


@functools.partial(
    pl.kernel,
    out_type=jax.ShapeDtypeStruct((NC, NP, HH), F32),
    mesh=_mesh,
    compiler_params=_sc_params,
    scratch_types=[
        pltpu.VMEM((2, CH), jnp.int32),
        pltpu.VMEM((2, CH), jnp.int32),
        pltpu.VMEM((2, CH, HP), F32),
        pltpu.VMEM((2, CH, HP), F32),
        pltpu.VMEM((2, CH, HEADS * HH), jnp.bfloat16),
        pltpu.VMEM((2, CH, HH), F32),
        pltpu.VMEM((RPS, HH), F32),
        pltpu.VMEM_SHARED((NP, HH), F32),
        [pltpu.SemaphoreType.DMA] * 2,
        [pltpu.SemaphoreType.DMA] * 2,
    ],
)
def _sc_pass3(src_hbm, dst_hbm, ex_hbm, winv_hbm, hw2_hbm, mpart_out,
              srcv, dstv, exb, wvb, hrows, msgb, zb, acc, semi, semg):
    c, s, wid = _worker()
    _zero_shared(zb, (acc,), s)
    plsc.subcore_barrier()

    def base_of(ci):
        ci = lax.rem(ci, NCH3)
        return pl.multiple_of(s * EPS3 + ci * CH, CH)

    def issue_idx(b, ci):
        base = base_of(ci)
        pltpu.async_copy(src_hbm.at[pl.ds(base, CH)], srcv.at[b], semi[b])
        pltpu.async_copy(dst_hbm.at[pl.ds(base, CH)], dstv.at[b], semi[b])

    def wait_idx(b):
        pltpu.make_async_copy(src_hbm.at[pl.ds(0, CH)], srcv.at[b], semi[b]).wait()
        pltpu.make_async_copy(dst_hbm.at[pl.ds(0, CH)], dstv.at[b], semi[b]).wait()

    def issue_g(b, ci):
        base = base_of(ci)
        pltpu.async_copy(winv_hbm.at[dstv.at[b]], wvb.at[b], semg[b])
        pltpu.async_copy(hw2_hbm.at[c].at[srcv.at[b]], hrows.at[b], semg[b])
        pltpu.async_copy(ex_hbm.at[pl.ds(base, CH)], exb.at[b], semg[b])

    def wait_g(b):
        pltpu.make_async_copy(winv_hbm.at[dstv.at[b]], wvb.at[b], semg[b]).wait()
        pltpu.make_async_copy(hw2_hbm.at[c].at[srcv.at[b]], hrows.at[b], semg[b]).wait()
        pltpu.make_async_copy(ex_hbm.at[pl.ds(0, CH)], exb.at[b], semg[b]).wait()

    def compute(b):
        @plsc.parallel_loop(0, CH, 1, unroll=8)
        def ebody(e):
            wv = exb[b, e] * wvb[b, e]
            acc0 = jnp.zeros((16,), F32)
            acc1 = jnp.zeros((16,), F32)
            for h in range(HEADS):
                wh = wv[h]
                iv = plsc.bitcast(hrows[b, e, pl.ds(h * HH, HH)], jnp.int32)
                p0 = plsc.bitcast(jnp.left_shift(iv, 16), F32)
                p1 = plsc.bitcast(jnp.bitwise_and(iv, jnp.int32(-65536)), F32)
                acc0 = acc0 + wh * p0
                acc1 = acc1 + wh * p1
            msgb[b, e, pl.ds(0, 16)] = acc0
            msgb[b, e, pl.ds(16, 16)] = acc1
        pltpu.sync_copy(msgb.at[b], acc.at[dstv.at[b]], add=True)

    issue_idx(0, 0)
    issue_idx(1, 1)
    wait_idx(0)
    issue_g(0, 0)

    def pair(i, _):
        a_ci = 2 * i
        wait_idx(1)
        issue_g(1, a_ci + 1)
        wait_g(0)
        compute(0)
        issue_idx(0, a_ci + 2)
        wait_idx(0)
        issue_g(0, a_ci + 2)
        wait_g(1)
        compute(1)
        issue_idx(1, a_ci + 3)
        return 0

    lax.fori_loop(0, NCH3 // 2, pair, 0)
    wait_idx(1)
    wait_g(0)
    plsc.subcore_barrier()
    off = pl.multiple_of(s * RPS, 8)
    pltpu.sync_copy(acc.at[pl.ds(off, RPS)], mpart_out.at[c, pl.ds(off, RPS)])


def _gelu(x):
    return 0.5 * x * (1.0 + lax.erf(x * 0.7071067811865476))


def _bn(x, g, b):
    m = jnp.mean(x, axis=0, keepdims=True)
    v = jnp.mean((x - m) ** 2, axis=0, keepdims=True)
    return (x - m) / jnp.sqrt(v + EPS) * g + b


def _emb_body(x_ref, w_ref, b_ref, o_ref):
    o_ref[...] = x_ref[...] @ w_ref[...] + b_ref[...]


def _ae_body(ea_ref, w_ref, o0_ref, o1_ref, o2_ref):
    ea = ea_ref[...]
    w = w_ref[...]
    o0_ref[...] = ea @ w[:, 0:HP]
    o1_ref[...] = ea @ w[:, HP:2 * HP]
    o2_ref[...] = ea @ w[:, 2 * HP:3 * HP]


def _pre_body(h_ref, ws_ref, wd_ref, wl_ref, as_ref, ad_ref, hwa_ref, hwb_ref):
    h = h_ref[...]
    as_ref[...] = h @ ws_ref[...]
    ad_ref[...] = h @ wd_ref[...]
    hw = h @ wl_ref[...]
    for h_i in range(HEADS):
        c0 = h_i * HID
        hwa_ref[:, h_i * HH:(h_i + 1) * HH] = hw[:, c0:c0 + HH].astype(jnp.bfloat16)
        hwb_ref[:, h_i * HH:(h_i + 1) * HH] = hw[:, c0 + HH:c0 + HID].astype(jnp.bfloat16)


def _loopattr_body(ls_ref, cnt_ref, o_ref):
    ls = ls_ref[0] + ls_ref[1]
    cn = cnt_ref[0] + cnt_ref[1]
    o_ref[...] = ls / jnp.maximum(cn, 1.0)


def _mid_body(la_ref, we_ref, as_ref, ad_ref, dp_ref, h_ref, wl_ref, wi_ref, sm_ref):
    ael = la_ref[...] @ we_ref[...]
    a = as_ref[...] + ad_ref[...] + ael
    a = jnp.where(a > 0.0, a, 0.2 * a)
    exs = jnp.exp(a)
    den = dp_ref[0] + dp_ref[1] + exs
    wi = 0.1 / den
    wi_ref[...] = wi
    wself = exs * wi
    hw = h_ref[...] @ wl_ref[...]
    acc = wself[:, 0:1] * hw[:, 0:HID]
    for h in range(1, HEADS):
        acc = acc + wself[:, h:h + 1] * hw[:, h * HID:(h + 1) * HID]
    sm_ref[...] = acc


def _post_body(mp_ref, perm_ref, sm_ref, b_ref, g_ref, be_ref, o_ref):
    hc = jnp.concatenate([mp_ref[0], mp_ref[1]], axis=1) @ perm_ref[...]
    hc = hc + sm_ref[...] + b_ref[...]
    hc = _gelu(hc)
    o_ref[...] = _bn(hc, g_ref[...], be_ref[...])


def _final_body(h_ref, w_ref, b_ref, g_ref, be_ref, batch_ref, o_ref):
    hf = h_ref[...] @ w_ref[...] + b_ref[...]
    hf = _gelu(hf)
    hf = _bn(hf, g_ref[...], be_ref[...])
    gids = lax.broadcasted_iota(jnp.int32, (N, NG), 1)
    oh = (batch_ref[...] == gids).astype(F32)
    s = lax.dot_general(oh, hf, (((0,), (0,)), ((), ())))
    cnt = lax.dot_general(oh, jnp.ones((N, 1), F32), (((0,), (0,)), ((), ())))
    o_ref[...] = s / jnp.maximum(cnt, 1.0)


def _row_block(nrows, cols, nblk):
    return pl.BlockSpec((nrows // nblk, cols), lambda i: (i, 0))


def _full(shape):
    return pl.BlockSpec(shape, lambda i: tuple(0 for _ in shape))


def _pad_heads(w):
    return jnp.pad(w, ((0, 0), (0, HP - w.shape[1])))


def kernel(x, edge_index, edge_attr, batch, W_emb, b_emb, lin_W0, lin_W1,
           lin_W2, att_src, att_dst, att_edge, lin_edge_W, gat_bias, bn_gamma,
           bn_beta, W_lin, b_lin, bn2_gamma, bn2_beta):
    src = edge_index[0]
    dst = edge_index[1]
    lins = [lin_W0, lin_W1, lin_W2]

    w_src_p, w_dst_p, w_edge_p = [], [], []
    for i in range(3):
        d = lins[i].shape[0]
        lw = lins[i].reshape(d, HEADS, HID)
        w_src_p.append(_pad_heads(jnp.einsum('dhk,hk->dh', lw, att_src[i])))
        w_dst_p.append(_pad_heads(jnp.einsum('dhk,hk->dh', lw, att_dst[i])))
        lwe = lin_edge_W[i].reshape(DE, HEADS, HID)
        w_edge_p.append(_pad_heads(jnp.einsum('dhk,hk->dh', lwe, att_edge[i])))
    w_edge_cat = jnp.concatenate(w_edge_p, axis=1)

    pm = np.zeros((HID, HID), np.float32)
    for col in range(HID):
        half, r = divmod(col, 2 * HH // 2)
        half, r = divmod(col, 32)
        sub, j = divmod(r, 16)
        pm[col, half * 32 + 2 * j + sub] = 1.0
    p64 = jnp.asarray(pm)

    emb = pl.pallas_call(
        _emb_body,
        grid=(10,),
        in_specs=[_row_block(N, 128, 10), _full((128, 10)), _full((1, 10))],
        out_specs=_row_block(N, 10, 10),
        out_shape=jax.ShapeDtypeStruct((N, 10), F32),
    )(x, W_emb, b_emb.reshape(1, -1))

    ae = pl.pallas_call(
        _ae_body,
        grid=(40,),
        in_specs=[_row_block(E, DE, 40), _full((DE, 3 * HP))],
        out_specs=[_row_block(E, HP, 40)] * 3,
        out_shape=[jax.ShapeDtypeStruct((E, HP), F32)] * 3,
    )(edge_attr, w_edge_cat)

    ls_p, cnt_p = _sc_phase0(dst, edge_attr)
    ls_p, cnt_p = ls_p[:, :N], cnt_p[:, :N]

    loop_attr = pl.pallas_call(
        _loopattr_body,
        grid=(1,),
        in_specs=[_full((NC, N, DE))] * 2,
        out_specs=_full((N, DE)),
        out_shape=jax.ShapeDtypeStruct((N, DE), F32),
    )(ls_p, cnt_p)

    h = emb
    for i in range(3):
        d = h.shape[1]
        asn, adn, hwa, hwb = pl.pallas_call(
            _pre_body,
            grid=(10,),
            in_specs=[_row_block(N, d, 10), _full((d, HP)), _full((d, HP)),
                      _full((d, HEADS * HID))],
            out_specs=[_row_block(N, HP, 10), _row_block(N, HP, 10),
                       _row_block(N, HEADS * HH, 10),
                       _row_block(N, HEADS * HH, 10)],
            out_shape=[jax.ShapeDtypeStruct((N, HP), F32),
                       jax.ShapeDtypeStruct((N, HP), F32),
                       jax.ShapeDtypeStruct((N, HEADS * HH), jnp.bfloat16),
                       jax.ShapeDtypeStruct((N, HEADS * HH), jnp.bfloat16)],
        )(h, w_src_p[i], w_dst_p[i], lins[i])
        hw2 = jnp.stack([hwa, hwb], axis=0)

        ex, dpart = _sc_pass2(src, dst, asn, adn, ae[i])
        dpart = dpart[:, :N]

        winv, selfmsg = pl.pallas_call(
            _mid_body,
            grid=(10,),
            in_specs=[_row_block(N, DE, 10), _full((DE, HP)),
                      _row_block(N, HP, 10), _row_block(N, HP, 10),
                      pl.BlockSpec((NC, N // 10, HP), lambda i: (0, i, 0)),
                      _row_block(N, d, 10), _full((d, HEADS * HID))],
            out_specs=[_row_block(N, HP, 10), _row_block(N, HID, 10)],
            out_shape=[jax.ShapeDtypeStruct((N, HP), F32),
                       jax.ShapeDtypeStruct((N, HID), F32)],
        )(loop_attr, w_edge_p[i], asn, adn, dpart, h, lins[i])

        mpart = _sc_pass3(src, dst, ex, winv, hw2)[:, :N]

        hc = pl.pallas_call(
            _post_body,
            grid=(1,),
            in_specs=[_full((NC, N, HH)), _full((HID, HID)), _full((N, HID)),
                      _full((1, HID)), _full((1, HID)), _full((1, HID))],
            out_specs=_full((N, HID)),
            out_shape=jax.ShapeDtypeStruct((N, HID), F32),
        )(mpart, p64, selfmsg, gat_bias[i].reshape(1, -1),
          bn_gamma[i].reshape(1, -1), bn_beta[i].reshape(1, -1))
        h = jnp.concatenate([h, hc], axis=1)

    dfin = h.shape[1]
    dpad = 256
    h_pad = jnp.pad(h, ((0, 0), (0, dpad - dfin)))
    w_pad = jnp.pad(W_lin, ((0, dpad - dfin), (0, 0)))
    out = pl.pallas_call(
        _final_body,
        grid=(1,),
        in_specs=[_full((N, dpad)), _full((dpad, 128)), _full((1, 128)),
                  _full((1, 128)), _full((1, 128)), _full((N, 1))],
        out_specs=_full((NG, 128)),
        out_shape=jax.ShapeDtypeStruct((NG, 128), F32),
    )(h_pad, w_pad, b_lin.reshape(1, -1), bn2_gamma.reshape(1, -1),
      bn2_beta.reshape(1, -1), batch.reshape(-1, 1))
    return out


# speedup vs baseline: 1.1322x; 1.1322x over previous
import functools

import jax
import jax.numpy as jnp
import numpy as np
from jax import lax
from jax.experimental import pallas as pl
from jax.experimental.pallas import tpu as pltpu
from jax.experimental.pallas import tpu_sc as plsc

N = 10000
E = 320000
HEADS = 10
HID = 64
HP = 16
DE = 16
NG = 64
EPS = 1e-5
F32 = jnp.float32

NC = 2
NS = 16
NW = NC * NS
EPW = E // NW
CH = 80
NCHUNK = EPW // CH
NP = 10240
RPS = NP // NS

_mesh = plsc.VectorSubcoreMesh(
    core_axis_name="c", subcore_axis_name="s", num_cores=NC, num_subcores=NS)
_sc_params = pltpu.CompilerParams(use_tc_tiling_on_sc=False, needs_layout_passes=False)


def _worker():
    c = lax.axis_index("c")
    s = lax.axis_index("s")
    return c, s, s * NC + c


def _zero_shared(zb, accs, s):
    z16 = jnp.zeros((16,), F32)

    def zbody(i, _):
        w = zb.shape[1]
        for c0 in range(0, w, 16):
            zb[i, pl.ds(c0, 16)] = z16
        return 0

    lax.fori_loop(0, RPS, zbody, 0)
    for acc in accs:
        pltpu.sync_copy(zb, acc.at[pl.ds(pl.multiple_of(s * RPS, 8), RPS)])


CH0 = 40
NCH0 = EPW // CH0


@functools.partial(
    pl.kernel,
    out_type=(jax.ShapeDtypeStruct((NC, NP, DE), F32),
              jax.ShapeDtypeStruct((NC, NP, DE), F32)),
    mesh=_mesh,
    compiler_params=_sc_params,
    scratch_types=[
        pltpu.VMEM((2, CH0), jnp.int32),
        pltpu.VMEM((2, CH0, DE), F32),
        pltpu.VMEM((CH0, DE), F32),
        pltpu.VMEM((RPS, DE), F32),
        pltpu.VMEM_SHARED((NP, DE), F32),
        pltpu.VMEM_SHARED((NP, DE), F32),
        [pltpu.SemaphoreType.DMA] * 2,
    ],
)
def _sc_phase0(dst_hbm, ea_hbm, ls_out, cnt_out,
               dstv, eabuf, onesb, zb, acc_a, acc_b, semg):
    c, s, wid = _worker()
    one16 = jnp.ones((16,), F32)

    def obody(i, _):
        onesb[i] = one16
        return 0

    lax.fori_loop(0, CH0, obody, 0)
    _zero_shared(zb, (acc_a, acc_b), s)
    plsc.subcore_barrier()

    def base_of(ci):
        ci = lax.rem(ci, NCH0)
        return pl.multiple_of(wid * EPW + ci * CH0, CH0)

    def issue(b, ci):
        base = base_of(ci)
        pltpu.async_copy(dst_hbm.at[pl.ds(base, CH0)], dstv.at[b], semg[b])
        pltpu.async_copy(ea_hbm.at[pl.ds(base, CH0)], eabuf.at[b], semg[b])

    def wait(b):
        pltpu.make_async_copy(dst_hbm.at[pl.ds(0, CH0)], dstv.at[b], semg[b]).wait()
        pltpu.make_async_copy(ea_hbm.at[pl.ds(0, CH0)], eabuf.at[b], semg[b]).wait()

    def scat(b):
        pltpu.sync_copy(eabuf.at[b], acc_a.at[dstv.at[b]], add=True)
        pltpu.sync_copy(onesb, acc_b.at[dstv.at[b]], add=True)

    issue(0, 0)
    issue(1, 1)

    def pair(i, _):
        wait(0)
        scat(0)
        issue(0, 2 * i + 2)
        wait(1)
        scat(1)
        issue(1, 2 * i + 3)
        return 0

    lax.fori_loop(0, NCH0 // 2, pair, 0)
    wait(0)
    wait(1)
    plsc.subcore_barrier()
    off = pl.multiple_of(s * RPS, 8)
    pltpu.sync_copy(acc_a.at[pl.ds(off, RPS)], ls_out.at[c, pl.ds(off, RPS)])
    pltpu.sync_copy(acc_b.at[pl.ds(off, RPS)], cnt_out.at[c, pl.ds(off, RPS)])


CH2 = 40
NCH2 = EPW // CH2


@functools.partial(
    pl.kernel,
    out_type=(jax.ShapeDtypeStruct((E, HP), F32),
              jax.ShapeDtypeStruct((NC, NP, HP), F32)),
    mesh=_mesh,
    compiler_params=_sc_params,
    scratch_types=[
        pltpu.VMEM((2, CH2), jnp.int32),
        pltpu.VMEM((2, CH2), jnp.int32),
        pltpu.VMEM((2, CH2, HP), F32),
        pltpu.VMEM((2, CH2, HP), F32),
        pltpu.VMEM((2, CH2, HP), F32),
        pltpu.VMEM((2, CH2, HP), F32),
        pltpu.VMEM((RPS, HP), F32),
        pltpu.VMEM_SHARED((NP, HP), F32),
        [pltpu.SemaphoreType.DMA] * 2,
        [pltpu.SemaphoreType.DMA] * 2,
        [pltpu.SemaphoreType.DMA] * 2,
    ],
)
def _sc_pass2(src_hbm, dst_hbm, asn_hbm, adn_hbm, ae_hbm, ex_out, dpart_out,
              srcv, dstv, asb, adb, aeb, exb, zb, acc, semi, semg, semw):
    c, s, wid = _worker()
    _zero_shared(zb, (acc,), s)
    plsc.subcore_barrier()

    def base_of(ci):
        ci = lax.rem(ci, NCH2)
        return pl.multiple_of(wid * EPW + ci * CH2, CH2)

    def issue_idx(b, ci):
        base = base_of(ci)
        pltpu.async_copy(src_hbm.at[pl.ds(base, CH2)], srcv.at[b], semi[b])
        pltpu.async_copy(dst_hbm.at[pl.ds(base, CH2)], dstv.at[b], semi[b])

    def wait_idx(b):
        pltpu.make_async_copy(src_hbm.at[pl.ds(0, CH2)], srcv.at[b], semi[b]).wait()
        pltpu.make_async_copy(dst_hbm.at[pl.ds(0, CH2)], dstv.at[b], semi[b]).wait()

    def issue_g(b, ci):
        base = base_of(ci)
        pltpu.async_copy(asn_hbm.at[srcv.at[b]], asb.at[b], semg[b])
        pltpu.async_copy(adn_hbm.at[dstv.at[b]], adb.at[b], semg[b])
        pltpu.async_copy(ae_hbm.at[pl.ds(base, CH2)], aeb.at[b], semg[b])

    def wait_g(b):
        pltpu.make_async_copy(asn_hbm.at[srcv.at[b]], asb.at[b], semg[b]).wait()
        pltpu.make_async_copy(adn_hbm.at[dstv.at[b]], adb.at[b], semg[b]).wait()
        pltpu.make_async_copy(ae_hbm.at[pl.ds(0, CH2)], aeb.at[b], semg[b]).wait()

    def wait_w(b):
        pltpu.make_async_copy(exb.at[b], ex_out.at[pl.ds(0, CH2)], semw[b]).wait()

    def compute(b, ci):
        wait_w(b)

        @plsc.parallel_loop(0, CH2, 1, unroll=4)
        def ebody(e):
            a = asb[b, e] + adb[b, e] + aeb[b, e]
            a = jnp.where(a > 0.0, a, 0.2 * a)
            exb[b, e] = jnp.exp(a)
        base = base_of(ci)
        pltpu.async_copy(exb.at[b], ex_out.at[pl.ds(base, CH2)], semw[b])
        pltpu.sync_copy(exb.at[b], acc.at[dstv.at[b]], add=True)

    for b in range(2):
        pltpu.async_copy(ae_hbm.at[pl.ds(0, CH2)], exb.at[b], semw[b])
    issue_idx(0, 0)
    issue_idx(1, 1)
    wait_idx(0)
    issue_g(0, 0)

    def pair(i, _):
        a_ci = 2 * i
        wait_idx(1)
        issue_g(1, a_ci + 1)
        wait_g(0)
        compute(0, a_ci)
        issue_idx(0, a_ci + 2)
        wait_idx(0)
        issue_g(0, a_ci + 2)
        wait_g(1)
        compute(1, a_ci + 1)
        issue_idx(1, a_ci + 3)
        return 0

    lax.fori_loop(0, NCH2 // 2, pair, 0)
    wait_idx(1)
    wait_g(0)
    wait_w(0)
    wait_w(1)
    plsc.subcore_barrier()
    off = pl.multiple_of(s * RPS, 8)
    pltpu.sync_copy(acc.at[pl.ds(off, RPS)], dpart_out.at[c, pl.ds(off, RPS)])


HH = HID // 2
EPS3 = E // NS
NCH3 = EPS3 // CH


@functools.partial(
    pl.kernel,
    out_type=jax.ShapeDtypeStruct((NC, NP, HH), F32),
    mesh=_mesh,
    compiler_params=_sc_params,
    scratch_types=[
        pltpu.VMEM((2, CH), jnp.int32),
        pltpu.VMEM((2, CH), jnp.int32),
        pltpu.VMEM((2, CH, HP), F32),
        pltpu.VMEM((2, CH, HP), F32),
        pltpu.VMEM((2, CH, HEADS * HH), jnp.bfloat16),
        pltpu.VMEM((2, CH, HH), F32),
        pltpu.VMEM((RPS, HH), F32),
        pltpu.VMEM_SHARED((NP, HH), F32),
        [pltpu.SemaphoreType.DMA] * 2,
        [pltpu.SemaphoreType.DMA] * 2,
    ],
)
def _sc_pass3(src_hbm, dst_hbm, ex_hbm, winv_hbm, hw2_hbm, mpart_out,
              srcv, dstv, exb, wvb, hrows, msgb, zb, acc, semi, semg):
    c, s, wid = _worker()
    _zero_shared(zb, (acc,), s)
    plsc.subcore_barrier()

    def base_of(ci):
        ci = lax.rem(ci, NCH3)
        return pl.multiple_of(s * EPS3 + ci * CH, CH)

    def issue_idx(b, ci):
        base = base_of(ci)
        pltpu.async_copy(src_hbm.at[pl.ds(base, CH)], srcv.at[b], semi[b])
        pltpu.async_copy(dst_hbm.at[pl.ds(base, CH)], dstv.at[b], semi[b])

    def wait_idx(b):
        pltpu.make_async_copy(src_hbm.at[pl.ds(0, CH)], srcv.at[b], semi[b]).wait()
        pltpu.make_async_copy(dst_hbm.at[pl.ds(0, CH)], dstv.at[b], semi[b]).wait()

    def issue_g(b, ci):
        base = base_of(ci)
        pltpu.async_copy(winv_hbm.at[dstv.at[b]], wvb.at[b], semg[b])
        pltpu.async_copy(hw2_hbm.at[c].at[srcv.at[b]], hrows.at[b], semg[b])
        pltpu.async_copy(ex_hbm.at[pl.ds(base, CH)], exb.at[b], semg[b])

    def wait_g(b):
        pltpu.make_async_copy(winv_hbm.at[dstv.at[b]], wvb.at[b], semg[b]).wait()
        pltpu.make_async_copy(hw2_hbm.at[c].at[srcv.at[b]], hrows.at[b], semg[b]).wait()
        pltpu.make_async_copy(ex_hbm.at[pl.ds(0, CH)], exb.at[b], semg[b]).wait()

    def compute(b):
        @plsc.parallel_loop(0, CH, 1, unroll=4)
        def ebody(e):
            wv = exb[b, e] * wvb[b, e]
            acc0 = jnp.zeros((16,), F32)
            acc1 = jnp.zeros((16,), F32)
            for h in range(HEADS):
                wh = wv[h]
                iv = plsc.bitcast(hrows[b, e, pl.ds(h * HH, HH)], jnp.int32)
                p0 = plsc.bitcast(jnp.left_shift(iv, 16), F32)
                p1 = plsc.bitcast(jnp.bitwise_and(iv, jnp.int32(-65536)), F32)
                acc0 = acc0 + wh * p0
                acc1 = acc1 + wh * p1
            msgb[b, e, pl.ds(0, 16)] = acc0
            msgb[b, e, pl.ds(16, 16)] = acc1
        pltpu.sync_copy(msgb.at[b], acc.at[dstv.at[b]], add=True)

    issue_idx(0, 0)
    issue_idx(1, 1)
    wait_idx(0)
    issue_g(0, 0)

    def pair(i, _):
        a_ci = 2 * i
        wait_idx(1)
        issue_g(1, a_ci + 1)
        wait_g(0)
        compute(0)
        issue_idx(0, a_ci + 2)
        wait_idx(0)
        issue_g(0, a_ci + 2)
        wait_g(1)
        compute(1)
        issue_idx(1, a_ci + 3)
        return 0

    lax.fori_loop(0, NCH3 // 2, pair, 0)
    wait_idx(1)
    wait_g(0)
    plsc.subcore_barrier()
    off = pl.multiple_of(s * RPS, 8)
    pltpu.sync_copy(acc.at[pl.ds(off, RPS)], mpart_out.at[c, pl.ds(off, RPS)])


def _gelu(x):
    return 0.5 * x * (1.0 + lax.erf(x * 0.7071067811865476))


def _bn(x, g, b):
    m = jnp.mean(x, axis=0, keepdims=True)
    v = jnp.mean((x - m) ** 2, axis=0, keepdims=True)
    return (x - m) / jnp.sqrt(v + EPS) * g + b


def _emb_body(x_ref, w_ref, b_ref, o_ref):
    o_ref[...] = x_ref[...] @ w_ref[...] + b_ref[...]


def _ae_body(ea_ref, w_ref, o0_ref, o1_ref, o2_ref):
    ea = ea_ref[...]
    w = w_ref[...]
    o0_ref[...] = ea @ w[:, 0:HP]
    o1_ref[...] = ea @ w[:, HP:2 * HP]
    o2_ref[...] = ea @ w[:, 2 * HP:3 * HP]


def _pre_body(h_ref, ws_ref, wd_ref, wl_ref, as_ref, ad_ref, hwa_ref, hwb_ref):
    h = h_ref[...]
    as_ref[...] = h @ ws_ref[...]
    ad_ref[...] = h @ wd_ref[...]
    hw = h @ wl_ref[...]
    for h_i in range(HEADS):
        c0 = h_i * HID
        hwa_ref[:, h_i * HH:(h_i + 1) * HH] = hw[:, c0:c0 + HH].astype(jnp.bfloat16)
        hwb_ref[:, h_i * HH:(h_i + 1) * HH] = hw[:, c0 + HH:c0 + HID].astype(jnp.bfloat16)


def _loopattr_body(ls_ref, cnt_ref, o_ref):
    ls = ls_ref[0] + ls_ref[1]
    cn = cnt_ref[0] + cnt_ref[1]
    o_ref[...] = ls / jnp.maximum(cn, 1.0)


def _mid_body(la_ref, we_ref, as_ref, ad_ref, dp_ref, h_ref, wl_ref, wi_ref, sm_ref):
    ael = la_ref[...] @ we_ref[...]
    a = as_ref[...] + ad_ref[...] + ael
    a = jnp.where(a > 0.0, a, 0.2 * a)
    exs = jnp.exp(a)
    den = dp_ref[0] + dp_ref[1] + exs
    wi = 0.1 / den
    wi_ref[...] = wi
    wself = exs * wi
    hw = h_ref[...] @ wl_ref[...]
    acc = wself[:, 0:1] * hw[:, 0:HID]
    for h in range(1, HEADS):
        acc = acc + wself[:, h:h + 1] * hw[:, h * HID:(h + 1) * HID]
    sm_ref[...] = acc


def _post_body(mp_ref, perm_ref, sm_ref, b_ref, g_ref, be_ref, o_ref):
    hc = jnp.concatenate([mp_ref[0], mp_ref[1]], axis=1) @ perm_ref[...]
    hc = hc + sm_ref[...] + b_ref[...]
    hc = _gelu(hc)
    o_ref[...] = _bn(hc, g_ref[...], be_ref[...])


def _final_body(h_ref, w_ref, b_ref, g_ref, be_ref, batch_ref, o_ref):
    hf = h_ref[...] @ w_ref[...] + b_ref[...]
    hf = _gelu(hf)
    hf = _bn(hf, g_ref[...], be_ref[...])
    gids = lax.broadcasted_iota(jnp.int32, (N, NG), 1)
    oh = (batch_ref[...] == gids).astype(F32)
    s = lax.dot_general(oh, hf, (((0,), (0,)), ((), ())))
    cnt = lax.dot_general(oh, jnp.ones((N, 1), F32), (((0,), (0,)), ((), ())))
    o_ref[...] = s / jnp.maximum(cnt, 1.0)


def _row_block(nrows, cols, nblk):
    return pl.BlockSpec((nrows // nblk, cols), lambda i: (i, 0))


def _full(shape):
    return pl.BlockSpec(shape, lambda i: tuple(0 for _ in shape))


def _pad_heads(w):
    return jnp.pad(w, ((0, 0), (0, HP - w.shape[1])))


def kernel(x, edge_index, edge_attr, batch, W_emb, b_emb, lin_W0, lin_W1,
           lin_W2, att_src, att_dst, att_edge, lin_edge_W, gat_bias, bn_gamma,
           bn_beta, W_lin, b_lin, bn2_gamma, bn2_beta):
    src = edge_index[0]
    dst = edge_index[1]
    lins = [lin_W0, lin_W1, lin_W2]

    w_src_p, w_dst_p, w_edge_p = [], [], []
    for i in range(3):
        d = lins[i].shape[0]
        lw = lins[i].reshape(d, HEADS, HID)
        w_src_p.append(_pad_heads(jnp.einsum('dhk,hk->dh', lw, att_src[i])))
        w_dst_p.append(_pad_heads(jnp.einsum('dhk,hk->dh', lw, att_dst[i])))
        lwe = lin_edge_W[i].reshape(DE, HEADS, HID)
        w_edge_p.append(_pad_heads(jnp.einsum('dhk,hk->dh', lwe, att_edge[i])))
    w_edge_cat = jnp.concatenate(w_edge_p, axis=1)

    pm = np.zeros((HID, HID), np.float32)
    for col in range(HID):
        half, r = divmod(col, 2 * HH // 2)
        half, r = divmod(col, 32)
        sub, j = divmod(r, 16)
        pm[col, half * 32 + 2 * j + sub] = 1.0
    p64 = jnp.asarray(pm)

    emb = pl.pallas_call(
        _emb_body,
        grid=(10,),
        in_specs=[_row_block(N, 128, 10), _full((128, 10)), _full((1, 10))],
        out_specs=_row_block(N, 10, 10),
        out_shape=jax.ShapeDtypeStruct((N, 10), F32),
    )(x, W_emb, b_emb.reshape(1, -1))

    ae = pl.pallas_call(
        _ae_body,
        grid=(40,),
        in_specs=[_row_block(E, DE, 40), _full((DE, 3 * HP))],
        out_specs=[_row_block(E, HP, 40)] * 3,
        out_shape=[jax.ShapeDtypeStruct((E, HP), F32)] * 3,
    )(edge_attr, w_edge_cat)

    ls_p, cnt_p = _sc_phase0(dst, edge_attr)
    ls_p, cnt_p = ls_p[:, :N], cnt_p[:, :N]

    loop_attr = pl.pallas_call(
        _loopattr_body,
        grid=(1,),
        in_specs=[_full((NC, N, DE))] * 2,
        out_specs=_full((N, DE)),
        out_shape=jax.ShapeDtypeStruct((N, DE), F32),
    )(ls_p, cnt_p)

    h = emb
    for i in range(3):
        d = h.shape[1]
        asn, adn, hwa, hwb = pl.pallas_call(
            _pre_body,
            grid=(10,),
            in_specs=[_row_block(N, d, 10), _full((d, HP)), _full((d, HP)),
                      _full((d, HEADS * HID))],
            out_specs=[_row_block(N, HP, 10), _row_block(N, HP, 10),
                       _row_block(N, HEADS * HH, 10),
                       _row_block(N, HEADS * HH, 10)],
            out_shape=[jax.ShapeDtypeStruct((N, HP), F32),
                       jax.ShapeDtypeStruct((N, HP), F32),
                       jax.ShapeDtypeStruct((N, HEADS * HH), jnp.bfloat16),
                       jax.ShapeDtypeStruct((N, HEADS * HH), jnp.bfloat16)],
        )(h, w_src_p[i], w_dst_p[i], lins[i])
        hw2 = jnp.stack([hwa, hwb], axis=0)

        ex, dpart = _sc_pass2(src, dst, asn, adn, ae[i])
        dpart = dpart[:, :N]

        winv, selfmsg = pl.pallas_call(
            _mid_body,
            grid=(10,),
            in_specs=[_row_block(N, DE, 10), _full((DE, HP)),
                      _row_block(N, HP, 10), _row_block(N, HP, 10),
                      pl.BlockSpec((NC, N // 10, HP), lambda i: (0, i, 0)),
                      _row_block(N, d, 10), _full((d, HEADS * HID))],
            out_specs=[_row_block(N, HP, 10), _row_block(N, HID, 10)],
            out_shape=[jax.ShapeDtypeStruct((N, HP), F32),
                       jax.ShapeDtypeStruct((N, HID), F32)],
        )(loop_attr, w_edge_p[i], asn, adn, dpart, h, lins[i])

        mpart = _sc_pass3(src, dst, ex, winv, hw2)[:, :N]

        hc = pl.pallas_call(
            _post_body,
            grid=(1,),
            in_specs=[_full((NC, N, HH)), _full((HID, HID)), _full((N, HID)),
                      _full((1, HID)), _full((1, HID)), _full((1, HID))],
            out_specs=_full((N, HID)),
            out_shape=jax.ShapeDtypeStruct((N, HID), F32),
        )(mpart, p64, selfmsg, gat_bias[i].reshape(1, -1),
          bn_gamma[i].reshape(1, -1), bn_beta[i].reshape(1, -1))
        h = jnp.concatenate([h, hc], axis=1)

    dfin = h.shape[1]
    dpad = 256
    h_pad = jnp.pad(h, ((0, 0), (0, dpad - dfin)))
    w_pad = jnp.pad(W_lin, ((0, dpad - dfin), (0, 0)))
    out = pl.pallas_call(
        _final_body,
        grid=(1,),
        in_specs=[_full((N, dpad)), _full((dpad, 128)), _full((1, 128)),
                  _full((1, 128)), _full((1, 128)), _full((N, 1))],
        out_specs=_full((NG, 128)),
        out_shape=jax.ShapeDtypeStruct((NG, 128), F32),
    )(h_pad, w_pad, b_lin.reshape(1, -1), bn2_gamma.reshape(1, -1),
      bn2_beta.reshape(1, -1), batch.reshape(-1, 1))
    return out
